# baseline (speedup 1.0000x reference)
import functools

import jax
import jax.numpy as jnp
from jax import lax
from jax.experimental import pallas as pl
from jax.experimental.pallas import tpu as pltpu
from jax.experimental.pallas import tpu_sc as plsc


_PREC = lax.Precision.DEFAULT

_NC = 2
_NS = 16
_NW = _NC * _NS
_LANES = 16


def _mm(a, b):
    return jnp.dot(a, b, precision=_PREC, preferred_element_type=jnp.float32)


def _selector(x, af, im, sel_W1, sel_b1, sW2p, sb2p, tb=512):
    n, h = x.shape
    hq = sel_W1.shape[1]
    nb = n // tb

    def body(x_ref, af_ref, im_ref, sW1_ref, sb1_ref, sW2_ref, sb2_ref,
             sel_ref, rank_ref, cnt_ref, acc_ref):
        i = pl.program_id(0)

        @pl.when(i == 0)
        def _():
            acc_ref[...] = jnp.zeros_like(acc_ref)

        x_blk = x_ref[...]
        half = x_blk.shape[1] // 2
        comb_a = x_blk[:, :half] * af_ref[...]
        comb_b = x_blk[:, half:] * im_ref[...]
        hsel = _mm(comb_a, sW1_ref[:half, :]) + _mm(comb_b, sW1_ref[half:, :])
        hsel = jnp.maximum(hsel + sb1_ref[...], 0.0)
        logits = _mm(hsel, sW2_ref[...]) + sb2_ref[...]
        l0 = logits[:, 0:1]
        l1 = logits[:, 1:2]
        l2 = logits[:, 2:3]
        sel1 = l1 > l0
        sel2 = l2 > jnp.maximum(l0, l1)
        e = jnp.where(sel2, 2, jnp.where(sel1, 1, 0)).astype(jnp.int32)
        sel_ref[...] = e

        lane = lax.broadcasted_iota(jnp.int32, (tb, 128), 1)
        onehot = (lane == e).astype(jnp.float32)
        row_i = lax.broadcasted_iota(jnp.int32, (tb, tb), 0)
        col_i = lax.broadcasted_iota(jnp.int32, (tb, tb), 1)
        tri = (col_i <= row_i).astype(jnp.float32)
        pref = _mm(tri, onehot)
        acc = acc_ref[...]
        rank = jnp.sum((pref + acc) * onehot, axis=1, keepdims=True)
        rank_ref[...] = rank.astype(jnp.int32)
        acc_new = acc + pref[tb - 1:tb, :]
        acc_ref[...] = acc_new
        cnt_ref[...] = acc_new

    return pl.pallas_call(
        body,
        grid=(nb,),
        in_specs=[
            pl.BlockSpec((tb, h), lambda i: (i, 0)),
            pl.BlockSpec((tb, 1), lambda i: (i, 0)),
            pl.BlockSpec((tb, 1), lambda i: (i, 0)),
            pl.BlockSpec((h, hq), lambda i: (0, 0)),
            pl.BlockSpec((hq,), lambda i: (0,)),
            pl.BlockSpec((hq, 128), lambda i: (0, 0)),
            pl.BlockSpec((128,), lambda i: (0,)),
        ],
        out_specs=[
            pl.BlockSpec((tb, 1), lambda i: (i, 0)),
            pl.BlockSpec((tb, 1), lambda i: (i, 0)),
            pl.BlockSpec((1, 128), lambda i: (0, 0)),
        ],
        out_shape=[
            jax.ShapeDtypeStruct((n, 1), jnp.int32),
            jax.ShapeDtypeStruct((n, 1), jnp.int32),
            jax.ShapeDtypeStruct((1, 128), jnp.float32),
        ],
        scratch_shapes=[pltpu.VMEM((1, 128), jnp.float32)],
    )(x, af, im, sel_W1, sel_b1, sW2p, sb2p)


def _permute_fwd(x, sel, rank, o1v, o2v, ch=16):
    n, h = x.shape
    rows_per_tile = n // _NW
    nchunk = rows_per_tile // ch
    mesh = plsc.VectorSubcoreMesh(core_axis_name="c", subcore_axis_name="s")

    @functools.partial(
        pl.kernel,
        out_type=jax.ShapeDtypeStruct((n, h), jnp.float32),
        mesh=mesh,
        scratch_types=[pltpu.VMEM((ch,), jnp.int32),
                       pltpu.VMEM((ch,), jnp.int32),
                       pltpu.VMEM((ch,), jnp.int32),
                       pltpu.VMEM((ch,), jnp.int32),
                       pltpu.VMEM((_LANES,), jnp.int32),
                       pltpu.VMEM((_LANES,), jnp.int32),
                       pltpu.VMEM((ch, h), jnp.float32),
                       pltpu.VMEM((ch, h), jnp.float32),
                       pltpu.SemaphoreType.DMA,
                       pltpu.SemaphoreType.DMA],
    )
    def k(x_hbm, sel_hbm, rank_hbm, o1_hbm, o2_hbm, xp_hbm,
          sel_s0, sel_s1, idx0, idx1, o1_s, o2_s, rows0, rows1, s0, s1):
        wid = lax.axis_index("s") * _NC + lax.axis_index("c")
        base = wid * rows_per_tile
        pltpu.sync_copy(o1_hbm, o1_s)
        pltpu.sync_copy(o2_hbm, o2_s)
        o1 = o1_s[...]
        o2 = o2_s[...]
        sels = (sel_s0, sel_s1)
        idxs = (idx0, idx1)
        rows = (rows0, rows1)
        sems = (s0, s1)
        wb = [None, None]
        for c in range(nchunk):
            sl = c % 2
            if wb[sl] is not None:
                wb[sl].wait()
            off = base + c * ch
            pltpu.sync_copy(sel_hbm.at[pl.ds(off, ch)], sels[sl])
            pltpu.sync_copy(rank_hbm.at[pl.ds(off, ch)], idxs[sl])
            v = sels[sl][...]
            dest = idxs[sl][...] - 1 + (v & 1) * o1 + (v >> 1) * o2
            idxs[sl][...] = dest
            pltpu.sync_copy(x_hbm.at[pl.ds(off, ch)], rows[sl])
            wb[sl] = pltpu.async_copy(rows[sl], xp_hbm.at[idxs[sl]], sems[sl])
        for hnd in wb:
            if hnd is not None:
                hnd.wait()

    return k(x, sel, rank, o1v, o2v)


def _permute_inv(yp, sel, rank, o1v, o2v, ch=16):
    n, h = yp.shape
    rows_per_tile = n // _NW
    nchunk = rows_per_tile // ch
    mesh = plsc.VectorSubcoreMesh(core_axis_name="c", subcore_axis_name="s")

    @functools.partial(
        pl.kernel,
        out_type=jax.ShapeDtypeStruct((n, h), jnp.float32),
        mesh=mesh,
        scratch_types=[pltpu.VMEM((ch,), jnp.int32),
                       pltpu.VMEM((ch,), jnp.int32),
                       pltpu.VMEM((ch,), jnp.int32),
                       pltpu.VMEM((ch,), jnp.int32),
                       pltpu.VMEM((_LANES,), jnp.int32),
                       pltpu.VMEM((_LANES,), jnp.int32),
                       pltpu.VMEM((ch, h), jnp.float32),
                       pltpu.VMEM((ch, h), jnp.float32),
                       pltpu.SemaphoreType.DMA,
                       pltpu.SemaphoreType.DMA,
                       pltpu.SemaphoreType.DMA,
                       pltpu.SemaphoreType.DMA],
    )
    def k(yp_hbm, sel_hbm, rank_hbm, o1_hbm, o2_hbm, out_hbm,
          sel_s0, sel_s1, idx0, idx1, o1_s, o2_s, rows0, rows1,
          g0, g1, w0, w1):
        wid = lax.axis_index("s") * _NC + lax.axis_index("c")
        base = wid * rows_per_tile
        pltpu.sync_copy(o1_hbm, o1_s)
        pltpu.sync_copy(o2_hbm, o2_s)
        o1 = o1_s[...]
        o2 = o2_s[...]
        sels = (sel_s0, sel_s1)
        idxs = (idx0, idx1)
        rows = (rows0, rows1)
        gsem = (g0, g1)
        wsem = (w0, w1)
        wb = [None, None]
        for c in range(nchunk):
            sl = c % 2
            if wb[sl] is not None:
                wb[sl].wait()
            off = base + c * ch
            pltpu.sync_copy(sel_hbm.at[pl.ds(off, ch)], sels[sl])
            pltpu.sync_copy(rank_hbm.at[pl.ds(off, ch)], idxs[sl])
            v = sels[sl][...]
            dest = idxs[sl][...] - 1 + (v & 1) * o1 + (v >> 1) * o2
            idxs[sl][...] = dest
            pltpu.async_copy(yp_hbm.at[idxs[sl]], rows[sl], gsem[sl]).wait()
            wb[sl] = pltpu.async_copy(rows[sl], out_hbm.at[pl.ds(off, ch)],
                                      wsem[sl])
        for hnd in wb:
            if hnd is not None:
                hnd.wait()

    return k(yp, sel, rank, o1v, o2v)


def _experts(xp, offs,
             comp_W1, comp_b1, adapt_W1, adapt_b1, decomp_W1, decomp_b1,
             comp_W2, comp_b2, adapt_W2, adapt_b2, decomp_W2, decomp_b2,
             tb=256):
    n, h = xp.shape
    nb = n // tb

    def body(offs_ref, xp_ref,
             cW1_ref, cb1_ref, aW1_ref, ab1_ref, dW1_ref, db1_ref,
             cW2_ref, cb2_ref, aW2_ref, ab2_ref, dW2_ref, db2_ref,
             out_ref):
        o1 = offs_ref[0]
        o2 = offs_ref[1]
        r0 = pl.program_id(0) * tb
        x_blk = xp_ref[...]
        out_ref[...] = x_blk
        ridx = lax.broadcasted_iota(jnp.int32, (tb, 1), 0) + r0

        @pl.when((r0 + tb > o1) & (r0 < o2))
        def _():
            d1 = _mm(_mm(_mm(x_blk, cW1_ref[...]) + cb1_ref[...],
                         aW1_ref[...]) + ab1_ref[...],
                     dW1_ref[...]) + db1_ref[...]
            m = (ridx >= o1) & (ridx < o2)
            out_ref[...] = jnp.where(m, d1, out_ref[...])

        @pl.when(r0 + tb > o2)
        def _():
            d2 = _mm(_mm(_mm(x_blk, cW2_ref[...]) + cb2_ref[...],
                         aW2_ref[...]) + ab2_ref[...],
                     dW2_ref[...]) + db2_ref[...]
            out_ref[...] = jnp.where(ridx >= o2, d2, out_ref[...])

    grid_spec = pltpu.PrefetchScalarGridSpec(
        num_scalar_prefetch=1,
        grid=(nb,),
        in_specs=[
            pl.BlockSpec((tb, h), lambda i, offs: (i, 0)),
            pl.BlockSpec((h, h // 2), lambda i, offs: (0, 0)),
            pl.BlockSpec((h // 2,), lambda i, offs: (0,)),
            pl.BlockSpec((h // 2, h // 2), lambda i, offs: (0, 0)),
            pl.BlockSpec((h // 2,), lambda i, offs: (0,)),
            pl.BlockSpec((h // 2, h), lambda i, offs: (0, 0)),
            pl.BlockSpec((h,), lambda i, offs: (0,)),
            pl.BlockSpec((h, h // 4), lambda i, offs: (0, 0)),
            pl.BlockSpec((h // 4,), lambda i, offs: (0,)),
            pl.BlockSpec((h // 4, h // 4), lambda i, offs: (0, 0)),
            pl.BlockSpec((h // 4,), lambda i, offs: (0,)),
            pl.BlockSpec((h // 4, h), lambda i, offs: (0, 0)),
            pl.BlockSpec((h,), lambda i, offs: (0,)),
        ],
        out_specs=pl.BlockSpec((tb, h), lambda i, offs: (i, 0)),
    )

    return pl.pallas_call(
        body,
        grid_spec=grid_spec,
        out_shape=jax.ShapeDtypeStruct((n, h), jnp.float32),
    )(offs, xp,
      comp_W1, comp_b1, adapt_W1, adapt_b1, decomp_W1, decomp_b1,
      comp_W2, comp_b2, adapt_W2, adapt_b2, decomp_W2, decomp_b2)


def kernel(hidden_states, access_frequency, importance_score,
           sel_W1, sel_b1, sel_W2, sel_b2,
           comp_W1, comp_b1, adapt_W1, adapt_b1, decomp_W1, decomp_b1,
           comp_W2, comp_b2, adapt_W2, adapt_b2, decomp_W2, decomp_b2):
    b, s, h = hidden_states.shape
    n = b * s
    x = hidden_states.reshape(n, h)
    af = access_frequency.reshape(n, 1)
    im = importance_score.reshape(n, 1)
    hq = sel_W1.shape[1]
    nl = sel_W2.shape[1]
    sW2p = jnp.zeros((hq, 128), sel_W2.dtype).at[:, :nl].set(sel_W2)
    sb2p = jnp.zeros((128,), sel_b2.dtype).at[:nl].set(sel_b2)

    sel2d, rank2d, counts = _selector(x, af, im, sel_W1, sel_b1, sW2p, sb2p)
    sel = sel2d.reshape(n)
    rank = rank2d.reshape(n)
    c0 = counts[0, 0].astype(jnp.int32)
    c1 = counts[0, 1].astype(jnp.int32)
    o1 = c0
    o2 = c0 + c1
    o1v = jnp.full((_LANES,), o1, jnp.int32)
    o2v = jnp.full((_LANES,), o2, jnp.int32)
    offs = jnp.stack([o1, o2])

    xp = _permute_fwd(x, sel, rank, o1v, o2v)
    yp = _experts(xp, offs,
                  comp_W1, comp_b1, adapt_W1, adapt_b1, decomp_W1, decomp_b1,
                  comp_W2, comp_b2, adapt_W2, adapt_b2, decomp_W2, decomp_b2)
    out = _permute_inv(yp, sel, rank, o1v, o2v)
    return out.reshape(b, s, h)


# speedup vs baseline: 1.4725x; 1.4725x over previous
import jax
import jax.numpy as jnp
from jax import lax
from jax.experimental import pallas as pl


_PREC = lax.Precision.DEFAULT


def kernel(hidden_states, access_frequency, importance_score,
           sel_W1, sel_b1, sel_W2, sel_b2,
           comp_W1, comp_b1, adapt_W1, adapt_b1, decomp_W1, decomp_b1,
           comp_W2, comp_b2, adapt_W2, adapt_b2, decomp_W2, decomp_b2):
    b, s, h = hidden_states.shape
    n = b * s
    x = hidden_states.reshape(n, h)
    af = access_frequency.reshape(n, 1)
    im = importance_score.reshape(n, 1)
    hq = sel_W1.shape[1]
    l = sel_W2.shape[1]
    sW2p = jnp.zeros((hq, 128), sel_W2.dtype).at[:, :l].set(sel_W2)
    sb2p = jnp.zeros((128,), sel_b2.dtype).at[:l].set(sel_b2)

    tb = 256
    nb = n // tb

    grid_spec = pl.GridSpec(
        grid=(nb,),
        in_specs=[
            pl.BlockSpec((tb, h), lambda i: (i, 0)),
            pl.BlockSpec((tb, 1), lambda i: (i, 0)),
            pl.BlockSpec((tb, 1), lambda i: (i, 0)),
            pl.BlockSpec((h, hq), lambda i: (0, 0)),
            pl.BlockSpec((hq,), lambda i: (0,)),
            pl.BlockSpec((hq, 128), lambda i: (0, 0)),
            pl.BlockSpec((128,), lambda i: (0,)),
            pl.BlockSpec((h, h // 2), lambda i: (0, 0)),
            pl.BlockSpec((h // 2,), lambda i: (0,)),
            pl.BlockSpec((h // 2, h // 2), lambda i: (0, 0)),
            pl.BlockSpec((h // 2,), lambda i: (0,)),
            pl.BlockSpec((h // 2, h), lambda i: (0, 0)),
            pl.BlockSpec((h,), lambda i: (0,)),
            pl.BlockSpec((h, h // 4), lambda i: (0, 0)),
            pl.BlockSpec((h // 4,), lambda i: (0,)),
            pl.BlockSpec((h // 4, h // 4), lambda i: (0, 0)),
            pl.BlockSpec((h // 4,), lambda i: (0,)),
            pl.BlockSpec((h // 4, h), lambda i: (0, 0)),
            pl.BlockSpec((h,), lambda i: (0,)),
        ],
        out_specs=pl.BlockSpec((tb, h), lambda i: (i, 0)),
    )

    def body(x_ref, af_ref, im_ref, sW1_ref, sb1_ref, sW2_ref, sb2_ref,
             cW1_ref, cb1_ref, aW1_ref, ab1_ref, dW1_ref, db1_ref,
             cW2_ref, cb2_ref, aW2_ref, ab2_ref, dW2_ref, db2_ref, out_ref):
        x_blk = x_ref[...]
        half = x_blk.shape[1] // 2
        comb_a = x_blk[:, :half] * af_ref[...]
        comb_b = x_blk[:, half:] * im_ref[...]

        def mm(a, bm):
            return jnp.dot(a, bm, precision=_PREC,
                           preferred_element_type=jnp.float32)

        hsel = mm(comb_a, sW1_ref[:half, :]) + mm(comb_b, sW1_ref[half:, :])
        hsel = jnp.maximum(hsel + sb1_ref[...], 0.0)
        logits = mm(hsel, sW2_ref[...]) + sb2_ref[...]
        l0 = logits[:, 0:1]
        l1 = logits[:, 1:2]
        l2 = logits[:, 2:3]
        sel1 = l1 > l0
        sel2 = l2 > jnp.maximum(l0, l1)

        def bmm(a, bm):
            return jnp.dot(a.astype(jnp.bfloat16), bm,
                           preferred_element_type=jnp.float32)

        d1 = bmm(bmm(bmm(x_blk, cW1_ref[...]) + cb1_ref[...],
                     aW1_ref[...]) + ab1_ref[...], dW1_ref[...]) + db1_ref[...]
        d2 = bmm(bmm(bmm(x_blk, cW2_ref[...]) + cb2_ref[...],
                     aW2_ref[...]) + ab2_ref[...], dW2_ref[...]) + db2_ref[...]
        out_ref[...] = jnp.where(sel2, d2, jnp.where(sel1, d1, x_blk))

    out = pl.pallas_call(
        body,
        grid_spec=grid_spec,
        out_shape=jax.ShapeDtypeStruct((n, h), jnp.float32),
    )(x, af, im, sel_W1, sel_b1, sW2p, sb2p,
      comp_W1.astype(jnp.bfloat16), comp_b1,
      adapt_W1.astype(jnp.bfloat16), adapt_b1,
      decomp_W1.astype(jnp.bfloat16), decomp_b1,
      comp_W2.astype(jnp.bfloat16), comp_b2,
      adapt_W2.astype(jnp.bfloat16), adapt_b2,
      decomp_W2.astype(jnp.bfloat16), decomp_b2)
    return out.reshape(b, s, h)


# speedup vs baseline: 1.7299x; 1.1748x over previous
import jax
import jax.numpy as jnp
from jax import lax
from jax.experimental import pallas as pl


_PREC = lax.Precision.DEFAULT


def kernel(hidden_states, access_frequency, importance_score,
           sel_W1, sel_b1, sel_W2, sel_b2,
           comp_W1, comp_b1, adapt_W1, adapt_b1, decomp_W1, decomp_b1,
           comp_W2, comp_b2, adapt_W2, adapt_b2, decomp_W2, decomp_b2):
    b, s, h = hidden_states.shape
    n = b * s
    x = hidden_states.reshape(n, h)
    af = access_frequency.reshape(n, 1)
    im = importance_score.reshape(n, 1)
    hq = sel_W1.shape[1]
    l = sel_W2.shape[1]
    sW2p = jnp.zeros((hq, 128), sel_W2.dtype).at[:, :l].set(sel_W2)
    sb2p = jnp.zeros((128,), sel_b2.dtype).at[:l].set(sel_b2)

    tb = 512
    nb = n // tb

    grid_spec = pl.GridSpec(
        grid=(nb,),
        in_specs=[
            pl.BlockSpec((tb, h), lambda i: (i, 0)),
            pl.BlockSpec((tb, 1), lambda i: (i, 0)),
            pl.BlockSpec((tb, 1), lambda i: (i, 0)),
            pl.BlockSpec((h, hq), lambda i: (0, 0)),
            pl.BlockSpec((hq,), lambda i: (0,)),
            pl.BlockSpec((hq, 128), lambda i: (0, 0)),
            pl.BlockSpec((128,), lambda i: (0,)),
            pl.BlockSpec((h, h // 2), lambda i: (0, 0)),
            pl.BlockSpec((h // 2,), lambda i: (0,)),
            pl.BlockSpec((h // 2, h // 2), lambda i: (0, 0)),
            pl.BlockSpec((h // 2,), lambda i: (0,)),
            pl.BlockSpec((h // 2, h), lambda i: (0, 0)),
            pl.BlockSpec((h,), lambda i: (0,)),
            pl.BlockSpec((h, h // 4), lambda i: (0, 0)),
            pl.BlockSpec((h // 4,), lambda i: (0,)),
            pl.BlockSpec((h // 4, h // 4), lambda i: (0, 0)),
            pl.BlockSpec((h // 4,), lambda i: (0,)),
            pl.BlockSpec((h // 4, h), lambda i: (0, 0)),
            pl.BlockSpec((h,), lambda i: (0,)),
        ],
        out_specs=pl.BlockSpec((tb, h), lambda i: (i, 0)),
    )

    def body(x_ref, af_ref, im_ref, sW1_ref, sb1_ref, sW2_ref, sb2_ref,
             cW1_ref, cb1_ref, aW1_ref, ab1_ref, dW1_ref, db1_ref,
             cW2_ref, cb2_ref, aW2_ref, ab2_ref, dW2_ref, db2_ref, out_ref):
        x_blk = x_ref[...]
        half = x_blk.shape[1] // 2
        comb_a = x_blk[:, :half] * af_ref[...]
        comb_b = x_blk[:, half:] * im_ref[...]

        def mm(a, bm):
            return jnp.dot(a, bm, precision=_PREC,
                           preferred_element_type=jnp.float32)

        hsel = mm(comb_a, sW1_ref[:half, :]) + mm(comb_b, sW1_ref[half:, :])
        hsel = jnp.maximum(hsel + sb1_ref[...], 0.0)
        logits = mm(hsel, sW2_ref[...]) + sb2_ref[...]
        l0 = logits[:, 0:1]
        l1 = logits[:, 1:2]
        l2 = logits[:, 2:3]
        sel1 = l1 > l0
        sel2 = l2 > jnp.maximum(l0, l1)

        d1 = mm(mm(mm(x_blk, cW1_ref[...]) + cb1_ref[...],
                   aW1_ref[...]) + ab1_ref[...], dW1_ref[...]) + db1_ref[...]
        d2 = mm(mm(mm(x_blk, cW2_ref[...]) + cb2_ref[...],
                   aW2_ref[...]) + ab2_ref[...], dW2_ref[...]) + db2_ref[...]
        out_ref[...] = jnp.where(sel2, d2, jnp.where(sel1, d1, x_blk))

    out = pl.pallas_call(
        body,
        grid_spec=grid_spec,
        out_shape=jax.ShapeDtypeStruct((n, h), jnp.float32),
    )(x, af, im, sel_W1, sel_b1, sW2p, sb2p,
      comp_W1, comp_b1, adapt_W1, adapt_b1, decomp_W1, decomp_b1,
      comp_W2, comp_b2, adapt_W2, adapt_b2, decomp_W2, decomp_b2)
    return out.reshape(b, s, h)
